# baseline (speedup 1.0000x reference)
import functools

import jax
import jax.numpy as jnp
from jax import lax
from jax.experimental import pallas as pl
from jax.experimental.pallas import tpu as pltpu
from jax.experimental.pallas import tpu_sc as plsc

_P = 0.9
_LAN = 128
_ROWS_PER_BLOCK = 8
_NSC_WORKERS = 32
_RADIX_BITS = 10
_NBINS = 1 << _RADIX_BITS
_KEY_MAX = (1 << 30) - 1


def _ceil_pow2(n):
    p = 1
    while p < n:
        p *= 2
    return p


_NSTRIPE = 4


def _sc_sort_chunk(src, dst, hist, off, chunk, desc):
    S = _NSTRIPE
    nvec = chunk // 16
    svec = nvec // S
    ones = jnp.ones((16,), jnp.int32)

    def key_of(v, desc_):
        k = plsc.bitcast(v, jnp.int32)
        return jnp.where(desc_ != 0, _KEY_MAX - k, k)

    bufs = [(src, dst), (dst, src), (src, dst)]
    for p in range(3):
        sh = p * _RADIX_BITS
        a, b = bufs[p]

        def zero_body(i, _):
            hist[pl.ds(i * 16, 16)] = jnp.zeros((16,), jnp.int32)
            return 0

        lax.fori_loop(0, S * _NBINS // 16, zero_body, 0)

        def hist_body(i, _):
            for s in range(S):
                v = a[pl.ds((s * svec + i) * 16, 16)]
                d = (key_of(v, desc) >> sh) & (_NBINS - 1)
                plsc.addupdate_scatter(hist, [d + s * _NBINS], ones)
            return 0

        lax.fori_loop(0, svec, hist_body, 0)

        def scan_body(i, carry):
            hh = [hist[pl.ds(s * _NBINS + i * 16, 16)] for s in range(S)]
            parts = [jnp.zeros((16,), jnp.int32)]
            for s in range(S):
                parts.append(parts[-1] + hh[s])
            tot = parts[S]
            cs = plsc.cumsum(tot)
            excl = cs - tot + carry
            for s in range(S):
                off[pl.ds(s * _NBINS + i * 16, 16)] = excl + parts[s]
            return carry + jnp.max(cs)

        lax.fori_loop(0, _NBINS // 16, scan_body, jnp.int32(0))

        def scat_body(i, _):
            for s in range(S):
                v = a[pl.ds((s * svec + i) * 16, 16)]
                d = (key_of(v, desc) >> sh) & (_NBINS - 1)
                cnt, last = plsc.scan_count(d)
                cnt = cnt.astype(jnp.int32)
                ds = d + s * _NBINS
                base = plsc.load_gather(off, [ds])
                plsc.store_scatter(b, [base + cnt - 1], v)
                plsc.addupdate_scatter(off, [ds], cnt, mask=last)
            return 0

        lax.fori_loop(0, svec, scat_body, 0)


def _sc_body(in_hbm, out_hbm, bufa, bufb, hist, off, *, rows, n_pad):
    chunk = n_pad // 4
    wid = lax.axis_index("s") * 2 + lax.axis_index("c")
    total = rows * 4
    per_w = total // _NSC_WORKERS

    def chunk_body(t, _):
        g = wid * per_w + t
        row = g // 4
        q = g % 4
        start = q * chunk
        pltpu.sync_copy(in_hbm.at[row, pl.ds(start, chunk)], bufa)
        desc = jnp.where((q & 1) == 0, 1, 0)
        _sc_sort_chunk(bufa, bufb, hist, off, chunk, desc)
        pltpu.sync_copy(bufb, out_hbm.at[row, pl.ds(start, chunk)])
        return 0

    lax.fori_loop(0, per_w, chunk_body, 0)


def _sc_sort(pk, rows, n_pad):
    chunk = n_pad // 4
    mesh = plsc.VectorSubcoreMesh(core_axis_name="c", subcore_axis_name="s",
                                  num_cores=2, num_subcores=16)
    f = pl.kernel(
        functools.partial(_sc_body, rows=rows, n_pad=n_pad),
        out_type=jax.ShapeDtypeStruct((rows, n_pad), jnp.float32),
        mesh=mesh,
        compiler_params=pltpu.CompilerParams(needs_layout_passes=False),
        scratch_types=[
            pltpu.VMEM((chunk,), jnp.float32),
            pltpu.VMEM((chunk,), jnp.float32),
            pltpu.VMEM((_NSTRIPE * _NBINS,), jnp.int32),
            pltpu.VMEM((_NSTRIPE * _NBINS,), jnp.int32),
        ],
    )
    return f(pk)


def _tc_body(s_ref, g_ref, p_ref, o_ref, *, sub, n_pad):
    R = _ROWS_PER_BLOCK
    x = s_ref[...]
    g = g_ref[...]

    iota_r = lax.broadcasted_iota(jnp.int32, (1, sub, 1), 1)
    iota_c = lax.broadcasted_iota(jnp.int32, (1, 1, _LAN), 2)

    for k in (n_pad // 2, n_pad):
        j = k // 2
        while j >= 1:
            if j >= _LAN:
                axis, d, pos = 1, j // _LAN, iota_r
            else:
                axis, d, pos = 2, j, iota_c
            up = jnp.roll(x, -d, axis=axis)
            dn = jnp.roll(x, d, axis=axis)
            is_lo = (pos & d) == 0
            part = jnp.where(is_lo, up, dn)
            if k >= n_pad:
                bitk_zero = jnp.full((1, 1, 1), True)
            elif k >= _LAN:
                bitk_zero = (iota_r & (k // _LAN)) == 0
            else:
                bitk_zero = (iota_c & k) == 0
            mx = jnp.maximum(x, part)
            mn = jnp.minimum(x, part)
            x = jnp.where(is_lo == bitk_zero, mx, mn)
            j //= 2

    cs = x
    sh = 1
    while sh < _LAN:
        z = jnp.zeros((R, sub, sh), jnp.float32)
        cs = cs + jnp.concatenate([z, cs[:, :, :-sh]], axis=2)
        sh *= 2
    tot = cs[:, :, _LAN - 1:_LAN]
    e = tot
    sh = 1
    while sh < sub:
        z = jnp.zeros((R, sh, 1), jnp.float32)
        e = e + jnp.concatenate([z, e[:, :-sh, :]], axis=1)
        sh *= 2
    cum = cs + (e - tot)

    first = (iota_r == 0) & (iota_c == 0)
    keep = (cum <= _P) | first
    norm = jnp.sum(jnp.where(keep, x, 0.0), axis=(1, 2), keepdims=True)
    norm = jnp.maximum(norm, 1e-9)
    logp = jnp.log(jnp.maximum(x / norm, 1e-20))
    score = jnp.where(keep, logp, -1e30) + g
    best = jnp.max(score, axis=(1, 2), keepdims=True)
    nidx = iota_r * _LAN + iota_c
    jstar = jnp.min(jnp.where(score == best, nidx, n_pad), axis=(1, 2),
                    keepdims=True)
    vstar = jnp.min(jnp.where(nidx == jstar, x, 2.0), axis=(1, 2),
                    keepdims=True)
    cg = jnp.sum((x > vstar).astype(jnp.int32), axis=(1, 2), keepdims=True)
    m = jstar - cg

    orig = p_ref[...]
    eq = orig == vstar
    mmax = jnp.max(m)

    def cond(carry):
        t, _ = carry
        return t <= mmax

    def body(carry):
        t, w = carry
        cand = jnp.where(eq & (nidx > w), nidx, n_pad)
        wnew = jnp.min(cand, axis=(1, 2), keepdims=True)
        w = jnp.where(t <= m, wnew, w)
        return t + 1, w

    _, w = lax.while_loop(cond, body,
                          (jnp.int32(0), jnp.full((R, 1, 1), -1, jnp.int32)))
    o_ref[0, 0, :] = w.reshape(R)


def kernel(probs):
    B, L, V = probs.shape
    rows = B * L
    R = _ROWS_PER_BLOCK
    n_pad = _ceil_pow2(V)
    sub = n_pad // _LAN
    flat = probs.reshape(rows, V)

    gum = jax.random.gumbel(jax.random.key(42), (rows, V), jnp.float32)

    pk = jnp.pad(flat, ((0, 0), (0, n_pad - V)))
    gk = jnp.pad(gum, ((0, 0), (0, n_pad - V))).reshape(rows, sub, _LAN)

    schunks = _sc_sort(pk, rows, n_pad)

    nblk = rows // R
    out = pl.pallas_call(
        functools.partial(_tc_body, sub=sub, n_pad=n_pad),
        grid=(nblk,),
        in_specs=[
            pl.BlockSpec((R, sub, _LAN), lambda i: (i, 0, 0)),
            pl.BlockSpec((R, sub, _LAN), lambda i: (i, 0, 0)),
            pl.BlockSpec((R, sub, _LAN), lambda i: (i, 0, 0)),
        ],
        out_specs=pl.BlockSpec((1, 1, R), lambda i: (i, 0, 0)),
        out_shape=jax.ShapeDtypeStruct((nblk, 1, R), jnp.int32),
    )(schunks.reshape(rows, sub, _LAN), gk, pk.reshape(rows, sub, _LAN))
    return out.reshape(B, L)


# speedup vs baseline: 1.0214x; 1.0214x over previous
import functools

import jax
import jax.numpy as jnp
from jax import lax
from jax.experimental import pallas as pl
from jax.experimental.pallas import tpu as pltpu
from jax.experimental.pallas import tpu_sc as plsc

_P = 0.9
_LAN = 128
_ROWS_PER_BLOCK = 8
_NSC_WORKERS = 32
_RADIX_BITS = 10
_NBINS = 1 << _RADIX_BITS
_KEY_MAX = (1 << 30) - 1


def _ceil_pow2(n):
    p = 1
    while p < n:
        p *= 2
    return p


_NSTRIPE = 4


def _sc_sort_chunk(src, dst, hists, offs, chunk, desc):
    S = _NSTRIPE
    nvec = chunk // 16
    svec = nvec // S
    ones = jnp.ones((16,), jnp.int32)

    def key_of(v, desc_):
        k = plsc.bitcast(v, jnp.int32)
        return jnp.where(desc_ != 0, _KEY_MAX - k, k)

    bufs = [(src, dst), (dst, src), (src, dst)]
    for p in range(3):
        sh = p * _RADIX_BITS
        a, b = bufs[p]

        def zero_body(i, _):
            for s in range(S):
                hists[s][pl.ds(i * 16, 16)] = jnp.zeros((16,), jnp.int32)
            return 0

        lax.fori_loop(0, _NBINS // 16, zero_body, 0)

        def hist_body(i, _):
            for s in range(S):
                v = a[pl.ds((s * svec + i) * 16, 16)]
                d = (key_of(v, desc) >> sh) & (_NBINS - 1)
                plsc.addupdate_scatter(hists[s], [d], ones)
            return 0

        lax.fori_loop(0, svec, hist_body, 0)

        def scan_body(i, carry):
            hh = [hists[s][pl.ds(i * 16, 16)] for s in range(S)]
            parts = [jnp.zeros((16,), jnp.int32)]
            for s in range(S):
                parts.append(parts[-1] + hh[s])
            tot = parts[S]
            cs = plsc.cumsum(tot)
            excl = cs - tot + carry
            for s in range(S):
                offs[s][pl.ds(i * 16, 16)] = excl + parts[s]
            return carry + jnp.max(cs)

        lax.fori_loop(0, _NBINS // 16, scan_body, jnp.int32(0))

        def scat_body(i, _):
            for s in range(S):
                v = a[pl.ds((s * svec + i) * 16, 16)]
                d = (key_of(v, desc) >> sh) & (_NBINS - 1)
                cnt, last = plsc.scan_count(d)
                cnt = cnt.astype(jnp.int32)
                base = plsc.load_gather(offs[s], [d])
                plsc.store_scatter(b, [base + cnt - 1], v)
                plsc.addupdate_scatter(offs[s], [d], cnt, mask=last)
            return 0

        lax.fori_loop(0, svec, scat_body, 0)


def _sc_body(in_hbm, out_hbm, bufa, bufb, *histoff, rows, n_pad):
    hists = histoff[:_NSTRIPE]
    offs = histoff[_NSTRIPE:]
    chunk = n_pad // 4
    wid = lax.axis_index("s") * 2 + lax.axis_index("c")
    total = rows * 4
    per_w = total // _NSC_WORKERS

    def chunk_body(t, _):
        g = wid * per_w + t
        row = g // 4
        q = g % 4
        start = q * chunk
        pltpu.sync_copy(in_hbm.at[row, pl.ds(start, chunk)], bufa)
        desc = jnp.where((q & 1) == 0, 1, 0)
        _sc_sort_chunk(bufa, bufb, hists, offs, chunk, desc)
        pltpu.sync_copy(bufb, out_hbm.at[row, pl.ds(start, chunk)])
        return 0

    lax.fori_loop(0, per_w, chunk_body, 0)


def _sc_sort(pk, rows, n_pad):
    chunk = n_pad // 4
    mesh = plsc.VectorSubcoreMesh(core_axis_name="c", subcore_axis_name="s",
                                  num_cores=2, num_subcores=16)
    f = pl.kernel(
        functools.partial(_sc_body, rows=rows, n_pad=n_pad),
        out_type=jax.ShapeDtypeStruct((rows, n_pad), jnp.float32),
        mesh=mesh,
        compiler_params=pltpu.CompilerParams(needs_layout_passes=False),
        scratch_types=(
            [pltpu.VMEM((chunk,), jnp.float32)] * 2
            + [pltpu.VMEM((_NBINS,), jnp.int32)] * (2 * _NSTRIPE)
        ),
    )
    return f(pk)


def _tc_body(s_ref, g_ref, p_ref, o_ref, *, sub, n_pad):
    R = _ROWS_PER_BLOCK
    x = s_ref[...]
    g = g_ref[...]

    iota_r = lax.broadcasted_iota(jnp.int32, (1, sub, 1), 1)
    iota_c = lax.broadcasted_iota(jnp.int32, (1, 1, _LAN), 2)

    for k in (n_pad // 2, n_pad):
        j = k // 2
        while j >= 1:
            if j >= _LAN:
                axis, d, pos = 1, j // _LAN, iota_r
            else:
                axis, d, pos = 2, j, iota_c
            up = jnp.roll(x, -d, axis=axis)
            dn = jnp.roll(x, d, axis=axis)
            is_lo = (pos & d) == 0
            part = jnp.where(is_lo, up, dn)
            if k >= n_pad:
                bitk_zero = jnp.full((1, 1, 1), True)
            elif k >= _LAN:
                bitk_zero = (iota_r & (k // _LAN)) == 0
            else:
                bitk_zero = (iota_c & k) == 0
            mx = jnp.maximum(x, part)
            mn = jnp.minimum(x, part)
            x = jnp.where(is_lo == bitk_zero, mx, mn)
            j //= 2

    cs = x
    sh = 1
    while sh < _LAN:
        z = jnp.zeros((R, sub, sh), jnp.float32)
        cs = cs + jnp.concatenate([z, cs[:, :, :-sh]], axis=2)
        sh *= 2
    tot = cs[:, :, _LAN - 1:_LAN]
    e = tot
    sh = 1
    while sh < sub:
        z = jnp.zeros((R, sh, 1), jnp.float32)
        e = e + jnp.concatenate([z, e[:, :-sh, :]], axis=1)
        sh *= 2
    cum = cs + (e - tot)

    first = (iota_r == 0) & (iota_c == 0)
    keep = (cum <= _P) | first
    norm = jnp.sum(jnp.where(keep, x, 0.0), axis=(1, 2), keepdims=True)
    norm = jnp.maximum(norm, 1e-9)
    logp = jnp.log(jnp.maximum(x / norm, 1e-20))
    score = jnp.where(keep, logp, -1e30) + g
    best = jnp.max(score, axis=(1, 2), keepdims=True)
    nidx = iota_r * _LAN + iota_c
    jstar = jnp.min(jnp.where(score == best, nidx, n_pad), axis=(1, 2),
                    keepdims=True)
    vstar = jnp.min(jnp.where(nidx == jstar, x, 2.0), axis=(1, 2),
                    keepdims=True)
    cg = jnp.sum((x > vstar).astype(jnp.int32), axis=(1, 2), keepdims=True)
    m = jstar - cg

    orig = p_ref[...]
    eq = orig == vstar
    mmax = jnp.max(m)

    def cond(carry):
        t, _ = carry
        return t <= mmax

    def body(carry):
        t, w = carry
        cand = jnp.where(eq & (nidx > w), nidx, n_pad)
        wnew = jnp.min(cand, axis=(1, 2), keepdims=True)
        w = jnp.where(t <= m, wnew, w)
        return t + 1, w

    _, w = lax.while_loop(cond, body,
                          (jnp.int32(0), jnp.full((R, 1, 1), -1, jnp.int32)))
    o_ref[0, 0, :] = w.reshape(R)


def kernel(probs):
    B, L, V = probs.shape
    rows = B * L
    R = _ROWS_PER_BLOCK
    n_pad = _ceil_pow2(V)
    sub = n_pad // _LAN
    flat = probs.reshape(rows, V)

    gum = jax.random.gumbel(jax.random.key(42), (rows, V), jnp.float32)

    pk = jnp.pad(flat, ((0, 0), (0, n_pad - V)))
    gk = jnp.pad(gum, ((0, 0), (0, n_pad - V))).reshape(rows, sub, _LAN)

    schunks = _sc_sort(pk, rows, n_pad)

    nblk = rows // R
    out = pl.pallas_call(
        functools.partial(_tc_body, sub=sub, n_pad=n_pad),
        grid=(nblk,),
        in_specs=[
            pl.BlockSpec((R, sub, _LAN), lambda i: (i, 0, 0)),
            pl.BlockSpec((R, sub, _LAN), lambda i: (i, 0, 0)),
            pl.BlockSpec((R, sub, _LAN), lambda i: (i, 0, 0)),
        ],
        out_specs=pl.BlockSpec((1, 1, R), lambda i: (i, 0, 0)),
        out_shape=jax.ShapeDtypeStruct((nblk, 1, R), jnp.int32),
    )(schunks.reshape(rows, sub, _LAN), gk, pk.reshape(rows, sub, _LAN))
    return out.reshape(B, L)


# speedup vs baseline: 1.1688x; 1.1443x over previous
import functools

import jax
import jax.numpy as jnp
from jax import lax
from jax.experimental import pallas as pl
from jax.experimental.pallas import tpu as pltpu
from jax.experimental.pallas import tpu_sc as plsc

_P = 0.9
_LAN = 128
_ROWS_PER_BLOCK = 8
_NSC_WORKERS = 32
_RADIX_BITS = 10
_NBINS = 1 << _RADIX_BITS
_KEY_MAX = (1 << 30) - 1


def _ceil_pow2(n):
    p = 1
    while p < n:
        p *= 2
    return p


_NSTRIPE = 4


def _sc_sort_chunk(src, dst, hists, offs, chunk, desc):
    S = _NSTRIPE
    nvec = chunk // 16
    svec = nvec // S
    ones = jnp.ones((16,), jnp.int32)

    def key_of(v, desc_):
        k = plsc.bitcast(v, jnp.int32)
        return jnp.where(desc_ != 0, _KEY_MAX - k, k)

    bufs = [(src, dst), (dst, src), (src, dst)]
    for p in range(3):
        sh = p * _RADIX_BITS
        a, b = bufs[p]

        def zero_body(i, _):
            for s in range(S):
                hists[s][pl.ds(i * 16, 16)] = jnp.zeros((16,), jnp.int32)
            return 0

        lax.fori_loop(0, _NBINS // 16, zero_body, 0)

        def hist_body(i, _):
            for s in range(S):
                v = a[pl.ds((s * svec + i) * 16, 16)]
                d = (key_of(v, desc) >> sh) & (_NBINS - 1)
                plsc.addupdate_scatter(hists[s], [d], ones)
            return 0

        lax.fori_loop(0, svec, hist_body, 0)

        def scan_body(i, carry):
            hh = [hists[s][pl.ds(i * 16, 16)] for s in range(S)]
            parts = [jnp.zeros((16,), jnp.int32)]
            for s in range(S):
                parts.append(parts[-1] + hh[s])
            tot = parts[S]
            cs = plsc.cumsum(tot)
            excl = cs - tot + carry
            for s in range(S):
                offs[s][pl.ds(i * 16, 16)] = excl + parts[s]
            return carry + jnp.max(cs)

        lax.fori_loop(0, _NBINS // 16, scan_body, jnp.int32(0))

        def scat_body(i, _):
            for s in range(S):
                v = a[pl.ds((s * svec + i) * 16, 16)]
                d = (key_of(v, desc) >> sh) & (_NBINS - 1)
                cnt, last = plsc.scan_count(d)
                cnt = cnt.astype(jnp.int32)
                base = plsc.load_gather(offs[s], [d])
                plsc.store_scatter(b, [base + cnt - 1], v)
                plsc.addupdate_scatter(offs[s], [d], cnt, mask=last)
            return 0

        lax.fori_loop(0, svec, scat_body, 0)


def _sc_body(in_hbm, out_hbm, bufa, bufb, *histoff, rows, n_pad):
    hists = histoff[:_NSTRIPE]
    offs = histoff[_NSTRIPE:]
    chunk = n_pad // 4
    wid = lax.axis_index("s") * 2 + lax.axis_index("c")
    total = rows * 4
    per_w = total // _NSC_WORKERS

    def chunk_body(t, _):
        g = wid * per_w + t
        row = g // 4
        q = g % 4
        start = q * chunk
        pltpu.sync_copy(in_hbm.at[row, pl.ds(start, chunk)], bufa)
        desc = jnp.where((q & 1) == 0, 1, 0)
        _sc_sort_chunk(bufa, bufb, hists, offs, chunk, desc)
        pltpu.sync_copy(bufb, out_hbm.at[row, pl.ds(start, chunk)])
        return 0

    lax.fori_loop(0, per_w, chunk_body, 0)


def _sc_sort(pk, rows, n_pad):
    chunk = n_pad // 4
    mesh = plsc.VectorSubcoreMesh(core_axis_name="c", subcore_axis_name="s",
                                  num_cores=2, num_subcores=16)
    f = pl.kernel(
        functools.partial(_sc_body, rows=rows, n_pad=n_pad),
        out_type=jax.ShapeDtypeStruct((rows, n_pad), jnp.float32),
        mesh=mesh,
        compiler_params=pltpu.CompilerParams(needs_layout_passes=False),
        scratch_types=(
            [pltpu.VMEM((chunk,), jnp.float32)] * 2
            + [pltpu.VMEM((_NBINS,), jnp.int32)] * (2 * _NSTRIPE)
        ),
    )
    return f(pk)


def _tc_full_body(p_ref, g_ref, o_ref, *, sub, n_pad):
    R = _ROWS_PER_BLOCK
    x = p_ref[...]
    g = g_ref[...]

    iota_r = lax.broadcasted_iota(jnp.int32, (1, sub, 1), 1)
    iota_c = lax.broadcasted_iota(jnp.int32, (1, 1, _LAN), 2)

    k = 2
    while k <= n_pad:
        j = k // 2
        while j >= 1:
            if j < sub:
                axis, d, pos = 1, j, iota_r
            else:
                axis, d, pos = 2, j // sub, iota_c
            up = jnp.roll(x, -d, axis=axis)
            dn = jnp.roll(x, d, axis=axis)
            is_lo = (pos & d) == 0
            part = jnp.where(is_lo, up, dn)
            if k >= n_pad:
                bitk_zero = jnp.full((1, 1, 1), True)
            elif k < sub:
                bitk_zero = (iota_r & k) == 0
            else:
                bitk_zero = (iota_c & (k // sub)) == 0
            mx = jnp.maximum(x, part)
            mn = jnp.minimum(x, part)
            x = jnp.where(is_lo == bitk_zero, mx, mn)
            j //= 2
        k *= 2

    cs = x
    sh = 1
    while sh < sub:
        z = jnp.zeros((R, sh, _LAN), jnp.float32)
        cs = cs + jnp.concatenate([z, cs[:, :-sh, :]], axis=1)
        sh *= 2
    tot = cs[:, sub - 1:sub, :]
    e = tot
    sh = 1
    while sh < _LAN:
        z = jnp.zeros((R, 1, sh), jnp.float32)
        e = e + jnp.concatenate([z, e[:, :, :-sh]], axis=2)
        sh *= 2
    cum = cs + (e - tot)

    first = (iota_r == 0) & (iota_c == 0)
    keep = (cum <= _P) | first
    norm = jnp.sum(jnp.where(keep, x, 0.0), axis=(1, 2), keepdims=True)
    norm = jnp.maximum(norm, 1e-9)
    logp = jnp.log(jnp.maximum(x / norm, 1e-20))
    score = jnp.where(keep, logp, -1e30) + g
    best = jnp.max(score, axis=(1, 2), keepdims=True)
    nidx = iota_c * sub + iota_r
    jstar = jnp.min(jnp.where(score == best, nidx, n_pad), axis=(1, 2),
                    keepdims=True)
    vstar = jnp.min(jnp.where(nidx == jstar, x, 2.0), axis=(1, 2),
                    keepdims=True)
    cg = jnp.sum((x > vstar).astype(jnp.int32), axis=(1, 2), keepdims=True)
    m = jstar - cg

    orig = p_ref[...]
    oidx = iota_r * _LAN + iota_c
    eq = orig == vstar
    mmax = jnp.max(m)

    def cond(carry):
        t, _ = carry
        return t <= mmax

    def body(carry):
        t, w = carry
        cand = jnp.where(eq & (oidx > w), oidx, n_pad)
        wnew = jnp.min(cand, axis=(1, 2), keepdims=True)
        w = jnp.where(t <= m, wnew, w)
        return t + 1, w

    _, w = lax.while_loop(cond, body,
                          (jnp.int32(0), jnp.full((R, 1, 1), -1, jnp.int32)))
    o_ref[0, 0, :] = w.reshape(R)


def _tc_body(s_ref, g_ref, p_ref, o_ref, *, sub, n_pad):
    R = _ROWS_PER_BLOCK
    x = s_ref[...]
    g = g_ref[...]

    iota_r = lax.broadcasted_iota(jnp.int32, (1, sub, 1), 1)
    iota_c = lax.broadcasted_iota(jnp.int32, (1, 1, _LAN), 2)

    for k in (n_pad // 2, n_pad):
        j = k // 2
        while j >= 1:
            if j >= _LAN:
                axis, d, pos = 1, j // _LAN, iota_r
            else:
                axis, d, pos = 2, j, iota_c
            up = jnp.roll(x, -d, axis=axis)
            dn = jnp.roll(x, d, axis=axis)
            is_lo = (pos & d) == 0
            part = jnp.where(is_lo, up, dn)
            if k >= n_pad:
                bitk_zero = jnp.full((1, 1, 1), True)
            elif k >= _LAN:
                bitk_zero = (iota_r & (k // _LAN)) == 0
            else:
                bitk_zero = (iota_c & k) == 0
            mx = jnp.maximum(x, part)
            mn = jnp.minimum(x, part)
            x = jnp.where(is_lo == bitk_zero, mx, mn)
            j //= 2

    cs = x
    sh = 1
    while sh < _LAN:
        z = jnp.zeros((R, sub, sh), jnp.float32)
        cs = cs + jnp.concatenate([z, cs[:, :, :-sh]], axis=2)
        sh *= 2
    tot = cs[:, :, _LAN - 1:_LAN]
    e = tot
    sh = 1
    while sh < sub:
        z = jnp.zeros((R, sh, 1), jnp.float32)
        e = e + jnp.concatenate([z, e[:, :-sh, :]], axis=1)
        sh *= 2
    cum = cs + (e - tot)

    first = (iota_r == 0) & (iota_c == 0)
    keep = (cum <= _P) | first
    norm = jnp.sum(jnp.where(keep, x, 0.0), axis=(1, 2), keepdims=True)
    norm = jnp.maximum(norm, 1e-9)
    logp = jnp.log(jnp.maximum(x / norm, 1e-20))
    score = jnp.where(keep, logp, -1e30) + g
    best = jnp.max(score, axis=(1, 2), keepdims=True)
    nidx = iota_r * _LAN + iota_c
    jstar = jnp.min(jnp.where(score == best, nidx, n_pad), axis=(1, 2),
                    keepdims=True)
    vstar = jnp.min(jnp.where(nidx == jstar, x, 2.0), axis=(1, 2),
                    keepdims=True)
    cg = jnp.sum((x > vstar).astype(jnp.int32), axis=(1, 2), keepdims=True)
    m = jstar - cg

    orig = p_ref[...]
    eq = orig == vstar
    mmax = jnp.max(m)

    def cond(carry):
        t, _ = carry
        return t <= mmax

    def body(carry):
        t, w = carry
        cand = jnp.where(eq & (nidx > w), nidx, n_pad)
        wnew = jnp.min(cand, axis=(1, 2), keepdims=True)
        w = jnp.where(t <= m, wnew, w)
        return t + 1, w

    _, w = lax.while_loop(cond, body,
                          (jnp.int32(0), jnp.full((R, 1, 1), -1, jnp.int32)))
    o_ref[0, 0, :] = w.reshape(R)


def _tc_full(pk, gp, rows, sub, n_pad):
    R = _ROWS_PER_BLOCK
    nblk = rows // R
    gk = gp.reshape(rows, _LAN, sub).swapaxes(1, 2)
    out = pl.pallas_call(
        functools.partial(_tc_full_body, sub=sub, n_pad=n_pad),
        grid=(nblk,),
        in_specs=[
            pl.BlockSpec((R, sub, _LAN), lambda i: (i, 0, 0)),
            pl.BlockSpec((R, sub, _LAN), lambda i: (i, 0, 0)),
        ],
        out_specs=pl.BlockSpec((1, 1, R), lambda i: (i, 0, 0)),
        out_shape=jax.ShapeDtypeStruct((nblk, 1, R), jnp.int32),
    )(pk.reshape(rows, sub, _LAN), gk)
    return out.reshape(rows)


def _tc_merge(schunks, pk, gp, rows, sub, n_pad):
    R = _ROWS_PER_BLOCK
    nblk = rows // R
    gk = gp.reshape(rows, sub, _LAN)
    out = pl.pallas_call(
        functools.partial(_tc_body, sub=sub, n_pad=n_pad),
        grid=(nblk,),
        in_specs=[
            pl.BlockSpec((R, sub, _LAN), lambda i: (i, 0, 0)),
            pl.BlockSpec((R, sub, _LAN), lambda i: (i, 0, 0)),
            pl.BlockSpec((R, sub, _LAN), lambda i: (i, 0, 0)),
        ],
        out_specs=pl.BlockSpec((1, 1, R), lambda i: (i, 0, 0)),
        out_shape=jax.ShapeDtypeStruct((nblk, 1, R), jnp.int32),
    )(schunks.reshape(rows, sub, _LAN), gk, pk.reshape(rows, sub, _LAN))
    return out.reshape(rows)


_TC_ROWS_FRAC_NUM = 5
_TC_ROWS_FRAC_DEN = 8


def kernel(probs):
    B, L, V = probs.shape
    rows = B * L
    R = _ROWS_PER_BLOCK
    n_pad = _ceil_pow2(V)
    sub = n_pad // _LAN
    flat = probs.reshape(rows, V)

    gum = jax.random.gumbel(jax.random.key(42), (rows, V), jnp.float32)

    pk = jnp.pad(flat, ((0, 0), (0, n_pad - V)))
    gp = jnp.pad(gum, ((0, 0), (0, n_pad - V)))

    r0 = (rows * _TC_ROWS_FRAC_NUM // _TC_ROWS_FRAC_DEN) // (8 * R) * (8 * R)
    if r0 == 0 or r0 == rows:
        schunks = _sc_sort(pk, rows, n_pad)
        out = _tc_merge(schunks, pk, gp, rows, sub, n_pad)
        return out.reshape(B, L)

    schunks = _sc_sort(pk[r0:], rows - r0, n_pad)
    out_head = _tc_full(pk[:r0], gp[:r0], r0, sub, n_pad)
    out_tail = _tc_merge(schunks, pk[r0:], gp[r0:], rows - r0, sub, n_pad)
    return jnp.concatenate([out_head, out_tail]).reshape(B, L)


# speedup vs baseline: 1.1689x; 1.0001x over previous
import functools

import jax
import jax.numpy as jnp
from jax import lax
from jax.experimental import pallas as pl
from jax.experimental.pallas import tpu as pltpu
from jax.experimental.pallas import tpu_sc as plsc

_P = 0.9
_LAN = 128
_ROWS_PER_BLOCK = 8
_NSC_WORKERS = 32
_RADIX_BITS = 10
_NBINS = 1 << _RADIX_BITS
_KEY_MAX = (1 << 30) - 1


def _ceil_pow2(n):
    p = 1
    while p < n:
        p *= 2
    return p


_NSTRIPE = 4


def _sc_sort_chunk(src, dst, hists, offs, chunk, desc):
    S = _NSTRIPE
    nvec = chunk // 16
    svec = nvec // S
    ones = jnp.ones((16,), jnp.int32)

    def key_of(v, desc_):
        k = plsc.bitcast(v, jnp.int32)
        return jnp.where(desc_ != 0, _KEY_MAX - k, k)

    bufs = [(src, dst), (dst, src), (src, dst)]
    for p in range(3):
        sh = p * _RADIX_BITS
        a, b = bufs[p]

        def zero_body(i, _):
            for s in range(S):
                hists[s][pl.ds(i * 16, 16)] = jnp.zeros((16,), jnp.int32)
            return 0

        lax.fori_loop(0, _NBINS // 16, zero_body, 0)

        def hist_body(i, _):
            for s in range(S):
                v = a[pl.ds((s * svec + i) * 16, 16)]
                d = (key_of(v, desc) >> sh) & (_NBINS - 1)
                plsc.addupdate_scatter(hists[s], [d], ones)
            return 0

        lax.fori_loop(0, svec, hist_body, 0)

        def scan_body(i, carry):
            hh = [hists[s][pl.ds(i * 16, 16)] for s in range(S)]
            parts = [jnp.zeros((16,), jnp.int32)]
            for s in range(S):
                parts.append(parts[-1] + hh[s])
            tot = parts[S]
            cs = plsc.cumsum(tot)
            excl = cs - tot + carry
            for s in range(S):
                offs[s][pl.ds(i * 16, 16)] = excl + parts[s]
            return carry + jnp.max(cs)

        lax.fori_loop(0, _NBINS // 16, scan_body, jnp.int32(0))

        def scat_body(i, _):
            for s in range(S):
                v = a[pl.ds((s * svec + i) * 16, 16)]
                d = (key_of(v, desc) >> sh) & (_NBINS - 1)
                cnt, last = plsc.scan_count(d)
                cnt = cnt.astype(jnp.int32)
                base = plsc.load_gather(offs[s], [d])
                plsc.store_scatter(b, [base + cnt - 1], v)
                plsc.addupdate_scatter(offs[s], [d], cnt, mask=last)
            return 0

        lax.fori_loop(0, svec, scat_body, 0)


def _sc_body(in_hbm, out_hbm, bufa, bufb, *histoff, rows, n_pad):
    hists = histoff[:_NSTRIPE]
    offs = histoff[_NSTRIPE:]
    chunk = n_pad // 4
    wid = lax.axis_index("s") * 2 + lax.axis_index("c")
    total = rows * 4
    per_w = total // _NSC_WORKERS

    def chunk_body(t, _):
        g = wid * per_w + t
        row = g // 4
        q = g % 4
        start = q * chunk
        pltpu.sync_copy(in_hbm.at[row, pl.ds(start, chunk)], bufa)
        desc = jnp.where((q & 1) == 0, 1, 0)
        _sc_sort_chunk(bufa, bufb, hists, offs, chunk, desc)
        pltpu.sync_copy(bufb, out_hbm.at[row, pl.ds(start, chunk)])
        return 0

    lax.fori_loop(0, per_w, chunk_body, 0)


def _sc_sort(pk, rows, n_pad):
    chunk = n_pad // 4
    mesh = plsc.VectorSubcoreMesh(core_axis_name="c", subcore_axis_name="s",
                                  num_cores=2, num_subcores=16)
    f = pl.kernel(
        functools.partial(_sc_body, rows=rows, n_pad=n_pad),
        out_type=jax.ShapeDtypeStruct((rows, n_pad), jnp.float32),
        mesh=mesh,
        compiler_params=pltpu.CompilerParams(needs_layout_passes=False),
        cost_estimate=pl.CostEstimate(
            flops=rows * n_pad * 30,
            bytes_accessed=rows * n_pad * 4 * 8,
            transcendentals=0,
        ),
        scratch_types=(
            [pltpu.VMEM((chunk,), jnp.float32)] * 2
            + [pltpu.VMEM((_NBINS,), jnp.int32)] * (2 * _NSTRIPE)
        ),
    )
    return f(pk)


def _tc_full_body(p_ref, g_ref, o_ref, *, sub, n_pad):
    R = _ROWS_PER_BLOCK
    x = p_ref[...]
    g = g_ref[...]

    iota_r = lax.broadcasted_iota(jnp.int32, (1, sub, 1), 1)
    iota_c = lax.broadcasted_iota(jnp.int32, (1, 1, _LAN), 2)

    k = 2
    while k <= n_pad:
        j = k // 2
        while j >= 1:
            if j < sub:
                axis, d, pos = 1, j, iota_r
            else:
                axis, d, pos = 2, j // sub, iota_c
            up = jnp.roll(x, -d, axis=axis)
            dn = jnp.roll(x, d, axis=axis)
            is_lo = (pos & d) == 0
            part = jnp.where(is_lo, up, dn)
            if k >= n_pad:
                bitk_zero = jnp.full((1, 1, 1), True)
            elif k < sub:
                bitk_zero = (iota_r & k) == 0
            else:
                bitk_zero = (iota_c & (k // sub)) == 0
            mx = jnp.maximum(x, part)
            mn = jnp.minimum(x, part)
            x = jnp.where(is_lo == bitk_zero, mx, mn)
            j //= 2
        k *= 2

    cs = x
    sh = 1
    while sh < sub:
        z = jnp.zeros((R, sh, _LAN), jnp.float32)
        cs = cs + jnp.concatenate([z, cs[:, :-sh, :]], axis=1)
        sh *= 2
    tot = cs[:, sub - 1:sub, :]
    e = tot
    sh = 1
    while sh < _LAN:
        z = jnp.zeros((R, 1, sh), jnp.float32)
        e = e + jnp.concatenate([z, e[:, :, :-sh]], axis=2)
        sh *= 2
    cum = cs + (e - tot)

    first = (iota_r == 0) & (iota_c == 0)
    keep = (cum <= _P) | first
    norm = jnp.sum(jnp.where(keep, x, 0.0), axis=(1, 2), keepdims=True)
    norm = jnp.maximum(norm, 1e-9)
    logp = jnp.log(jnp.maximum(x / norm, 1e-20))
    score = jnp.where(keep, logp, -1e30) + g
    best = jnp.max(score, axis=(1, 2), keepdims=True)
    nidx = iota_c * sub + iota_r
    jstar = jnp.min(jnp.where(score == best, nidx, n_pad), axis=(1, 2),
                    keepdims=True)
    vstar = jnp.min(jnp.where(nidx == jstar, x, 2.0), axis=(1, 2),
                    keepdims=True)
    cg = jnp.sum((x > vstar).astype(jnp.int32), axis=(1, 2), keepdims=True)
    m = jstar - cg

    orig = p_ref[...]
    oidx = iota_r * _LAN + iota_c
    eq = orig == vstar
    mmax = jnp.max(m)

    def cond(carry):
        t, _ = carry
        return t <= mmax

    def body(carry):
        t, w = carry
        cand = jnp.where(eq & (oidx > w), oidx, n_pad)
        wnew = jnp.min(cand, axis=(1, 2), keepdims=True)
        w = jnp.where(t <= m, wnew, w)
        return t + 1, w

    _, w = lax.while_loop(cond, body,
                          (jnp.int32(0), jnp.full((R, 1, 1), -1, jnp.int32)))
    o_ref[0, 0, :] = w.reshape(R)


def _tc_body(s_ref, g_ref, p_ref, o_ref, *, sub, n_pad):
    R = _ROWS_PER_BLOCK
    x = s_ref[...]
    g = g_ref[...]

    iota_r = lax.broadcasted_iota(jnp.int32, (1, sub, 1), 1)
    iota_c = lax.broadcasted_iota(jnp.int32, (1, 1, _LAN), 2)

    for k in (n_pad // 2, n_pad):
        j = k // 2
        while j >= 1:
            if j >= _LAN:
                axis, d, pos = 1, j // _LAN, iota_r
            else:
                axis, d, pos = 2, j, iota_c
            up = jnp.roll(x, -d, axis=axis)
            dn = jnp.roll(x, d, axis=axis)
            is_lo = (pos & d) == 0
            part = jnp.where(is_lo, up, dn)
            if k >= n_pad:
                bitk_zero = jnp.full((1, 1, 1), True)
            elif k >= _LAN:
                bitk_zero = (iota_r & (k // _LAN)) == 0
            else:
                bitk_zero = (iota_c & k) == 0
            mx = jnp.maximum(x, part)
            mn = jnp.minimum(x, part)
            x = jnp.where(is_lo == bitk_zero, mx, mn)
            j //= 2

    cs = x
    sh = 1
    while sh < _LAN:
        z = jnp.zeros((R, sub, sh), jnp.float32)
        cs = cs + jnp.concatenate([z, cs[:, :, :-sh]], axis=2)
        sh *= 2
    tot = cs[:, :, _LAN - 1:_LAN]
    e = tot
    sh = 1
    while sh < sub:
        z = jnp.zeros((R, sh, 1), jnp.float32)
        e = e + jnp.concatenate([z, e[:, :-sh, :]], axis=1)
        sh *= 2
    cum = cs + (e - tot)

    first = (iota_r == 0) & (iota_c == 0)
    keep = (cum <= _P) | first
    norm = jnp.sum(jnp.where(keep, x, 0.0), axis=(1, 2), keepdims=True)
    norm = jnp.maximum(norm, 1e-9)
    logp = jnp.log(jnp.maximum(x / norm, 1e-20))
    score = jnp.where(keep, logp, -1e30) + g
    best = jnp.max(score, axis=(1, 2), keepdims=True)
    nidx = iota_r * _LAN + iota_c
    jstar = jnp.min(jnp.where(score == best, nidx, n_pad), axis=(1, 2),
                    keepdims=True)
    vstar = jnp.min(jnp.where(nidx == jstar, x, 2.0), axis=(1, 2),
                    keepdims=True)
    cg = jnp.sum((x > vstar).astype(jnp.int32), axis=(1, 2), keepdims=True)
    m = jstar - cg

    orig = p_ref[...]
    eq = orig == vstar
    mmax = jnp.max(m)

    def cond(carry):
        t, _ = carry
        return t <= mmax

    def body(carry):
        t, w = carry
        cand = jnp.where(eq & (nidx > w), nidx, n_pad)
        wnew = jnp.min(cand, axis=(1, 2), keepdims=True)
        w = jnp.where(t <= m, wnew, w)
        return t + 1, w

    _, w = lax.while_loop(cond, body,
                          (jnp.int32(0), jnp.full((R, 1, 1), -1, jnp.int32)))
    o_ref[0, 0, :] = w.reshape(R)


def _tc_full(pk, gp, rows, sub, n_pad):
    R = _ROWS_PER_BLOCK
    nblk = rows // R
    gk = gp.reshape(rows, _LAN, sub).swapaxes(1, 2)
    out = pl.pallas_call(
        functools.partial(_tc_full_body, sub=sub, n_pad=n_pad),
        grid=(nblk,),
        in_specs=[
            pl.BlockSpec((R, sub, _LAN), lambda i: (i, 0, 0)),
            pl.BlockSpec((R, sub, _LAN), lambda i: (i, 0, 0)),
        ],
        out_specs=pl.BlockSpec((1, 1, R), lambda i: (i, 0, 0)),
        out_shape=jax.ShapeDtypeStruct((nblk, 1, R), jnp.int32),
    )(pk.reshape(rows, sub, _LAN), gk)
    return out.reshape(rows)


def _tc_merge(schunks, pk, gp, rows, sub, n_pad):
    R = _ROWS_PER_BLOCK
    nblk = rows // R
    gk = gp.reshape(rows, sub, _LAN)
    out = pl.pallas_call(
        functools.partial(_tc_body, sub=sub, n_pad=n_pad),
        grid=(nblk,),
        in_specs=[
            pl.BlockSpec((R, sub, _LAN), lambda i: (i, 0, 0)),
            pl.BlockSpec((R, sub, _LAN), lambda i: (i, 0, 0)),
            pl.BlockSpec((R, sub, _LAN), lambda i: (i, 0, 0)),
        ],
        out_specs=pl.BlockSpec((1, 1, R), lambda i: (i, 0, 0)),
        out_shape=jax.ShapeDtypeStruct((nblk, 1, R), jnp.int32),
    )(schunks.reshape(rows, sub, _LAN), gk, pk.reshape(rows, sub, _LAN))
    return out.reshape(rows)


_TC_ROWS_FRAC_NUM = 5
_TC_ROWS_FRAC_DEN = 8


def kernel(probs):
    B, L, V = probs.shape
    rows = B * L
    R = _ROWS_PER_BLOCK
    n_pad = _ceil_pow2(V)
    sub = n_pad // _LAN
    flat = probs.reshape(rows, V)

    gum = jax.random.gumbel(jax.random.key(42), (rows, V), jnp.float32)

    pk = jnp.pad(flat, ((0, 0), (0, n_pad - V)))
    gp = jnp.pad(gum, ((0, 0), (0, n_pad - V)))

    r0 = (rows * _TC_ROWS_FRAC_NUM // _TC_ROWS_FRAC_DEN) // (8 * R) * (8 * R)
    if r0 == 0 or r0 == rows:
        schunks = _sc_sort(pk, rows, n_pad)
        out = _tc_merge(schunks, pk, gp, rows, sub, n_pad)
        return out.reshape(B, L)

    schunks = _sc_sort(pk[r0:], rows - r0, n_pad)
    out_head = _tc_full(pk[:r0], gp[:r0], r0, sub, n_pad)
    out_tail = _tc_merge(schunks, pk[r0:], gp[r0:], rows - r0, sub, n_pad)
    return jnp.concatenate([out_head, out_tail]).reshape(B, L)


# speedup vs baseline: 1.3413x; 1.1474x over previous
import jax
import jax.numpy as jnp
from jax import lax
from jax.experimental import pallas as pl

_P = 0.9
_LAN = 128
_ROWS_PER_BLOCK = 8


def _ceil_pow2(n):
    p = 1
    while p < n:
        p *= 2
    return p


def _body(p_ref, g_ref, o_ref, *, sub, n_pad):
    R = _ROWS_PER_BLOCK
    x = p_ref[...]
    g = g_ref[...]

    iota_r = lax.broadcasted_iota(jnp.int32, (1, sub, 1), 1)
    iota_c = lax.broadcasted_iota(jnp.int32, (1, 1, _LAN), 2)

    k = 2
    while k <= n_pad:
        j = k // 2
        while j >= 1:
            if j < sub and j >= 8:
                d = j
                grp = sub // (2 * d)
                x4 = x.reshape(R, grp, 2, d, _LAN)
                lo = x4[:, :, 0]
                hi = x4[:, :, 1]
                mx = jnp.maximum(lo, hi)
                mn = jnp.minimum(lo, hi)
                if k >= n_pad:
                    bz = jnp.full((1, 1, 1, 1), True)
                elif k < sub:
                    iq = lax.broadcasted_iota(jnp.int32, (1, grp, 1, 1), 1)
                    bz = (iq & (k // (2 * d))) == 0
                else:
                    ic = lax.broadcasted_iota(jnp.int32, (1, 1, 1, _LAN), 3)
                    bz = (ic & (k // sub)) == 0
                nlo = jnp.where(bz, mx, mn)
                nhi = jnp.where(bz, mn, mx)
                x = jnp.concatenate([nlo[:, :, None], nhi[:, :, None]],
                                    axis=2).reshape(R, sub, _LAN)
                j //= 2
                continue
            if j < sub:
                axis, d, pos = 1, j, iota_r
            else:
                axis, d, pos = 2, j // sub, iota_c
            up = jnp.roll(x, -d, axis=axis)
            dn = jnp.roll(x, d, axis=axis)
            is_lo = (pos & d) == 0
            part = jnp.where(is_lo, up, dn)
            if k >= n_pad:
                bitk_zero = jnp.full((1, 1, 1), True)
            elif k < sub:
                bitk_zero = (iota_r & k) == 0
            else:
                bitk_zero = (iota_c & (k // sub)) == 0
            mx = jnp.maximum(x, part)
            mn = jnp.minimum(x, part)
            x = jnp.where(is_lo == bitk_zero, mx, mn)
            j //= 2
        k *= 2

    cs = x
    sh = 1
    while sh < sub:
        z = jnp.zeros((R, sh, _LAN), jnp.float32)
        cs = cs + jnp.concatenate([z, cs[:, :-sh, :]], axis=1)
        sh *= 2
    tot = cs[:, sub - 1:sub, :]
    e = tot
    sh = 1
    while sh < _LAN:
        z = jnp.zeros((R, 1, sh), jnp.float32)
        e = e + jnp.concatenate([z, e[:, :, :-sh]], axis=2)
        sh *= 2
    cum = cs + (e - tot)

    first = (iota_r == 0) & (iota_c == 0)
    keep = (cum <= _P) | first
    norm = jnp.sum(jnp.where(keep, x, 0.0), axis=(1, 2), keepdims=True)
    norm = jnp.maximum(norm, 1e-9)
    logp = jnp.log(jnp.maximum(x / norm, 1e-20))
    score = jnp.where(keep, logp, -1e30) + g
    best = jnp.max(score, axis=(1, 2), keepdims=True)
    nidx = iota_c * sub + iota_r
    jstar = jnp.min(jnp.where(score == best, nidx, n_pad), axis=(1, 2),
                    keepdims=True)
    vstar = jnp.min(jnp.where(nidx == jstar, x, 2.0), axis=(1, 2),
                    keepdims=True)
    cg = jnp.sum((x > vstar).astype(jnp.int32), axis=(1, 2), keepdims=True)
    m = jstar - cg

    orig = p_ref[...]
    oidx = iota_r * _LAN + iota_c
    eq = orig == vstar
    mmax = jnp.max(m)

    def cond(carry):
        t, _ = carry
        return t <= mmax

    def body(carry):
        t, w = carry
        cand = jnp.where(eq & (oidx > w), oidx, n_pad)
        wnew = jnp.min(cand, axis=(1, 2), keepdims=True)
        w = jnp.where(t <= m, wnew, w)
        return t + 1, w

    _, w = lax.while_loop(cond, body,
                          (jnp.int32(0), jnp.full((R, 1, 1), -1, jnp.int32)))
    o_ref[0, 0, :] = w.reshape(R)


def kernel(probs):
    B, L, V = probs.shape
    rows = B * L
    R = _ROWS_PER_BLOCK
    n_pad = _ceil_pow2(V)
    sub = n_pad // _LAN
    flat = probs.reshape(rows, V)

    gum = jax.random.gumbel(jax.random.key(42), (rows, V), jnp.float32)

    pp = jnp.pad(flat, ((0, 0), (0, n_pad - V)))
    gp = jnp.pad(gum, ((0, 0), (0, n_pad - V)))
    gk = gp.reshape(rows, _LAN, sub).swapaxes(1, 2)
    pk = pp.reshape(rows, sub, _LAN)

    nblk = rows // R
    import functools
    out = pl.pallas_call(
        functools.partial(_body, sub=sub, n_pad=n_pad),
        grid=(nblk,),
        in_specs=[
            pl.BlockSpec((R, sub, _LAN), lambda i: (i, 0, 0)),
            pl.BlockSpec((R, sub, _LAN), lambda i: (i, 0, 0)),
        ],
        out_specs=pl.BlockSpec((1, 1, R), lambda i: (i, 0, 0)),
        out_shape=jax.ShapeDtypeStruct((nblk, 1, R), jnp.int32),
    )(pk, gk)
    return out.reshape(B, L)
